# Initial kernel scaffold; baseline (speedup 1.0000x reference)
import jax
import jax.numpy as jnp
from jax.experimental import pallas as pl


def kernel(x, edge_index, W1, b1, W2, b2, W3, b3, Wc, bc, Wr, br):
    raise NotImplementedError("write your pallas kernel here")


# speedup vs baseline: 13.5769x; 13.5769x over previous
import functools

import jax
import jax.numpy as jnp
from jax import lax
from jax.experimental import pallas as pl
from jax.experimental.pallas import tpu as pltpu
from jax.experimental.pallas import tpu_sc as plsc

N = 10000
E = 320000
F_IN = 128
H = 64

NC = 2
NS = 16
NW = NC * NS
CHUNK = 128
CPW = 80
E_PAD = NW * CPW * CHUNK
TROWS = 632
N_SH = NS * TROWS

_mesh = plsc.VectorSubcoreMesh(core_axis_name="c", subcore_axis_name="s")
_sc_params = pltpu.CompilerParams(use_tc_tiling_on_sc=False)


@functools.partial(
    pl.kernel,
    out_type=jax.ShapeDtypeStruct((NC, N_SH, 16), jnp.float32),
    mesh=_mesh,
    scratch_types=[
        pltpu.VMEM((CPW, CHUNK), jnp.int32),
        pltpu.VMEM((CHUNK, 16), jnp.float32),
        pltpu.VMEM_SHARED((N_SH, 16), jnp.float32),
    ],
    compiler_params=_sc_params,
)
def _sc_degree(dst_hbm, ones_hbm, zero_hbm, out_hbm, dst_v, ones_v, acc_sh):
    cid = lax.axis_index("c")
    sid = lax.axis_index("s")
    wid = sid * NC + cid
    pltpu.sync_copy(zero_hbm, acc_sh.at[pl.ds(sid * TROWS, TROWS)])
    pltpu.sync_copy(dst_hbm.at[pl.ds(wid * CPW, CPW)], dst_v)
    pltpu.sync_copy(ones_hbm, ones_v)
    plsc.subcore_barrier()

    def body(j, carry):
        pltpu.sync_copy(ones_v, acc_sh.at[dst_v.at[j]], add=True)
        return carry

    lax.fori_loop(0, CPW, body, 0, unroll=4)
    plsc.subcore_barrier()
    pltpu.sync_copy(acc_sh.at[pl.ds(sid * TROWS, TROWS)],
                    out_hbm.at[cid, pl.ds(sid * TROWS, TROWS)])


@functools.partial(
    pl.kernel,
    out_type=jax.ShapeDtypeStruct((NC, N_SH, H), jnp.float32),
    mesh=_mesh,
    scratch_types=[
        pltpu.VMEM((CPW, CHUNK), jnp.int32),
        pltpu.VMEM((CPW, CHUNK), jnp.int32),
        pltpu.VMEM((CHUNK, H), jnp.float32),
        pltpu.VMEM((CHUNK, H), jnp.float32),
        pltpu.VMEM_SHARED((N_SH, H), jnp.float32),
        pltpu.SemaphoreType.DMA,
        pltpu.SemaphoreType.DMA,
    ],
    compiler_params=_sc_params,
)
def _sc_aggregate(g_hbm, src_hbm, dst_hbm, zero_hbm, out_hbm,
                  src_v, dst_v, buf_a, buf_b, acc_sh, sem_a, sem_b):
    cid = lax.axis_index("c")
    sid = lax.axis_index("s")
    wid = sid * NC + cid
    pltpu.sync_copy(zero_hbm, acc_sh.at[pl.ds(sid * TROWS, TROWS)])
    pltpu.sync_copy(src_hbm.at[pl.ds(wid * CPW, CPW)], src_v)
    pltpu.sync_copy(dst_hbm.at[pl.ds(wid * CPW, CPW)], dst_v)
    plsc.subcore_barrier()

    pltpu.async_copy(g_hbm.at[src_v.at[0]], buf_a, sem_a)

    def body(j, carry):
        @pl.when(j + 1 < CPW)
        def _():
            pltpu.async_copy(g_hbm.at[src_v.at[j + 1]], buf_b, sem_b)
        pltpu.make_async_copy(g_hbm.at[src_v.at[j]], buf_a, sem_a).wait()
        pltpu.sync_copy(buf_a, acc_sh.at[dst_v.at[j]], add=True)

        @pl.when(j + 2 < CPW)
        def _():
            pltpu.async_copy(g_hbm.at[src_v.at[j + 2]], buf_a, sem_a)
        pltpu.make_async_copy(g_hbm.at[src_v.at[j + 1]], buf_b, sem_b).wait()
        pltpu.sync_copy(buf_b, acc_sh.at[dst_v.at[j + 1]], add=True)
        return carry

    lax.fori_loop(0, CPW // 2, lambda i, c: body(i * 2, c), 0)
    plsc.subcore_barrier()
    pltpu.sync_copy(acc_sh.at[pl.ds(sid * TROWS, TROWS)],
                    out_hbm.at[cid, pl.ds(sid * TROWS, TROWS)])


_BLK = 1000


def _row_spec(width):
    return pl.BlockSpec((_BLK, width), lambda i: (i, 0))


def _part_spec(core, width):
    return pl.BlockSpec((1, _BLK, width), lambda i, c=core: (c, i, 0))


def _full_spec(shape):
    return pl.BlockSpec(shape, lambda i: (0,) * len(shape))


def _tc_prep_body(x_ref, w1_ref, d0_ref, d1_ref, g_ref, dinv_ref):
    deg = d0_ref[0, :, 0:1] + d1_ref[0, :, 0:1] + 1.0
    dinv = lax.rsqrt(deg)
    h = jnp.dot(x_ref[...], w1_ref[...], preferred_element_type=jnp.float32)
    g_ref[...] = h * dinv
    dinv_ref[...] = dinv


def _tc_prep(x, W1, degw):
    return pl.pallas_call(
        _tc_prep_body,
        grid=(N // _BLK,),
        in_specs=[_row_spec(F_IN), _full_spec((F_IN, H)),
                  _part_spec(0, 16), _part_spec(1, 16)],
        out_specs=[_row_spec(H), _row_spec(1)],
        out_shape=[jax.ShapeDtypeStruct((N, H), jnp.float32),
                   jax.ShapeDtypeStruct((N, 1), jnp.float32)],
    )(x, W1, degw, degw)


def _tc_mid_body(p0_ref, p1_ref, g_ref, dinv_ref, b_ref, w_ref, out_ref):
    dinv = dinv_ref[...]
    pre = (p0_ref[0] + p1_ref[0] + g_ref[...]) * dinv + b_ref[...]
    h = jnp.maximum(pre, 0.0)
    out_ref[...] = jnp.dot(h, w_ref[...],
                           preferred_element_type=jnp.float32) * dinv


def _tc_mid(p, g, dinv, b, Wn):
    return pl.pallas_call(
        _tc_mid_body,
        grid=(N // _BLK,),
        in_specs=[_part_spec(0, H), _part_spec(1, H), _row_spec(H),
                  _row_spec(1), _full_spec((1, H)), _full_spec((H, H))],
        out_specs=_row_spec(H),
        out_shape=jax.ShapeDtypeStruct((N, H), jnp.float32),
    )(p, p, g, dinv, b, Wn)


def _tc_final_body(p0_ref, p1_ref, g_ref, dinv_ref, b_ref, wc_ref, bc_ref,
                   wr_ref, br_ref, emb_ref, cls_ref, reg_ref):
    emb = (p0_ref[0] + p1_ref[0] + g_ref[...]) * dinv_ref[...] + b_ref[...]
    emb_ref[...] = emb
    zc = jnp.sum(emb * wc_ref[...], axis=1, keepdims=True) + bc_ref[...]
    cls_ref[...] = jax.nn.sigmoid(zc)
    reg_ref[...] = jnp.sum(emb * wr_ref[...], axis=1, keepdims=True) + br_ref[...]


def _tc_final(p, g, dinv, b3, Wc, bc, Wr, br):
    return pl.pallas_call(
        _tc_final_body,
        grid=(N // _BLK,),
        in_specs=[_part_spec(0, H), _part_spec(1, H), _row_spec(H),
                  _row_spec(1),
                  _full_spec((1, H)), _full_spec((1, H)), _full_spec((1, 1)),
                  _full_spec((1, H)), _full_spec((1, 1))],
        out_specs=[_row_spec(H), _row_spec(1), _row_spec(1)],
        out_shape=[jax.ShapeDtypeStruct((N, H), jnp.float32),
                   jax.ShapeDtypeStruct((N, 1), jnp.float32),
                   jax.ShapeDtypeStruct((N, 1), jnp.float32)],
    )(p, p, g, dinv, b3, Wc, bc, Wr, br)


def kernel(x, edge_index, W1, b1, W2, b2, W3, b3, Wc, bc, Wr, br):
    src = edge_index[0]
    dst = edge_index[1]
    pad = E_PAD - E
    srcp = jnp.concatenate([src, jnp.zeros((pad,), jnp.int32)])
    dstp = jnp.concatenate([dst, jnp.full((pad,), N, jnp.int32)])
    srcp = srcp.reshape(NW * CPW, CHUNK)
    dstp = dstp.reshape(NW * CPW, CHUNK)
    ones16 = jnp.ones((CHUNK, 16), jnp.float32)
    zero16 = jnp.zeros((TROWS, 16), jnp.float32)
    zero64 = jnp.zeros((TROWS, H), jnp.float32)

    degw = _sc_degree(dstp, ones16, zero16)
    g1, dinv = _tc_prep(x, W1, degw)
    p = _sc_aggregate(g1, srcp, dstp, zero64)
    g2 = _tc_mid(p, g1, dinv, b1.reshape(1, H), W2)
    p = _sc_aggregate(g2, srcp, dstp, zero64)
    g3 = _tc_mid(p, g2, dinv, b2.reshape(1, H), W3)
    p = _sc_aggregate(g3, srcp, dstp, zero64)
    emb, cls, reg = _tc_final(p, g3, dinv, b3.reshape(1, H),
                              Wc.reshape(1, H), bc.reshape(1, 1),
                              Wr.reshape(1, H), br.reshape(1, 1))
    return (emb, cls, reg)


# speedup vs baseline: 14.0274x; 1.0332x over previous
import functools

import jax
import jax.numpy as jnp
from jax import lax
from jax.experimental import pallas as pl
from jax.experimental.pallas import tpu as pltpu
from jax.experimental.pallas import tpu_sc as plsc

N = 10000
E = 320000
F_IN = 128
H = 64

NC = 2
NS = 16
NW = NC * NS
CHUNK = 128
CPW = 80
E_PAD = NW * CPW * CHUNK
TROWS = 632
N_SH = NS * TROWS

_mesh = plsc.VectorSubcoreMesh(core_axis_name="c", subcore_axis_name="s")
_sc_params = pltpu.CompilerParams(use_tc_tiling_on_sc=False)


@functools.partial(
    pl.kernel,
    out_type=jax.ShapeDtypeStruct((NC, N_SH, 16), jnp.float32),
    mesh=_mesh,
    scratch_types=[
        pltpu.VMEM((CPW, CHUNK), jnp.int32),
        pltpu.VMEM((CHUNK, 16), jnp.float32),
        pltpu.VMEM_SHARED((N_SH, 16), jnp.float32),
    ],
    compiler_params=_sc_params,
)
def _sc_degree(dst_hbm, ones_hbm, zero_hbm, out_hbm, dst_v, ones_v, acc_sh):
    cid = lax.axis_index("c")
    sid = lax.axis_index("s")
    wid = sid * NC + cid
    pltpu.sync_copy(zero_hbm, acc_sh.at[pl.ds(sid * TROWS, TROWS)])
    pltpu.sync_copy(dst_hbm.at[pl.ds(wid * CPW, CPW)], dst_v)
    pltpu.sync_copy(ones_hbm, ones_v)
    plsc.subcore_barrier()

    def body(j, carry):
        pltpu.sync_copy(ones_v, acc_sh.at[dst_v.at[j]], add=True)
        return carry

    lax.fori_loop(0, CPW, body, 0, unroll=4)
    plsc.subcore_barrier()
    pltpu.sync_copy(acc_sh.at[pl.ds(sid * TROWS, TROWS)],
                    out_hbm.at[cid, pl.ds(sid * TROWS, TROWS)])


@functools.partial(
    pl.kernel,
    out_type=jax.ShapeDtypeStruct((NC, N_SH, H), jnp.float32),
    mesh=_mesh,
    scratch_types=[
        pltpu.VMEM((CPW, CHUNK), jnp.int32),
        pltpu.VMEM((CPW, CHUNK), jnp.int32),
        pltpu.VMEM((CHUNK, H), jnp.float32),
        pltpu.VMEM((CHUNK, H), jnp.float32),
        pltpu.VMEM_SHARED((N_SH, H), jnp.float32),
        pltpu.SemaphoreType.DMA,
        pltpu.SemaphoreType.DMA,
    ],
    compiler_params=_sc_params,
)
def _sc_aggregate(g_hbm, src_hbm, dst_hbm, zero_hbm, out_hbm,
                  src_v, dst_v, buf_a, buf_b, acc_sh, sem_a, sem_b):
    cid = lax.axis_index("c")
    sid = lax.axis_index("s")
    wid = sid * NC + cid
    pltpu.sync_copy(zero_hbm, acc_sh.at[pl.ds(sid * TROWS, TROWS)])
    pltpu.sync_copy(src_hbm.at[pl.ds(wid * CPW, CPW)], src_v)
    pltpu.sync_copy(dst_hbm.at[pl.ds(wid * CPW, CPW)], dst_v)
    plsc.subcore_barrier()

    pltpu.async_copy(g_hbm.at[src_v.at[0]], buf_a, sem_a)

    def body(j, carry):
        @pl.when(j + 1 < CPW)
        def _():
            pltpu.async_copy(g_hbm.at[src_v.at[j + 1]], buf_b, sem_b)
        pltpu.make_async_copy(g_hbm.at[src_v.at[j]], buf_a, sem_a).wait()
        pltpu.sync_copy(buf_a, acc_sh.at[dst_v.at[j]], add=True)

        @pl.when(j + 2 < CPW)
        def _():
            pltpu.async_copy(g_hbm.at[src_v.at[j + 2]], buf_a, sem_a)
        pltpu.make_async_copy(g_hbm.at[src_v.at[j + 1]], buf_b, sem_b).wait()
        pltpu.sync_copy(buf_b, acc_sh.at[dst_v.at[j + 1]], add=True)
        return carry

    lax.fori_loop(0, CPW // 2, lambda i, c: body(i * 2, c), 0)
    plsc.subcore_barrier()
    pltpu.sync_copy(acc_sh.at[pl.ds(sid * TROWS, TROWS)],
                    out_hbm.at[cid, pl.ds(sid * TROWS, TROWS)])


_BLK = 1000


def _row_spec(width):
    return pl.BlockSpec((_BLK, width), lambda i: (i, 0))


def _part_spec(core, width):
    return pl.BlockSpec((1, _BLK, width), lambda i, c=core: (c, i, 0))


def _full_spec(shape):
    return pl.BlockSpec(shape, lambda i: (0,) * len(shape))


def _tc_prep_body(x_ref, w1_ref, d0_ref, d1_ref, g_ref, dinv_ref):
    deg = d0_ref[0, :, 0:1] + d1_ref[0, :, 0:1] + 1.0
    dinv = lax.rsqrt(deg)
    h = jnp.dot(x_ref[...], w1_ref[...], preferred_element_type=jnp.float32)
    g_ref[...] = h * dinv
    dinv_ref[...] = dinv


def _tc_prep(x, W1, degw):
    return pl.pallas_call(
        _tc_prep_body,
        grid=(N // _BLK,),
        in_specs=[_row_spec(F_IN), _full_spec((F_IN, H)),
                  _part_spec(0, 16), _part_spec(1, 16)],
        out_specs=[_row_spec(H), _row_spec(1)],
        out_shape=[jax.ShapeDtypeStruct((N, H), jnp.float32),
                   jax.ShapeDtypeStruct((N, 1), jnp.float32)],
    )(x, W1, degw, degw)


def _tc_mid_body(p0_ref, p1_ref, g_ref, dinv_ref, b_ref, w_ref, out_ref):
    dinv = dinv_ref[...]
    pre = (p0_ref[0] + p1_ref[0] + g_ref[...]) * dinv + b_ref[...]
    h = jnp.maximum(pre, 0.0)
    out_ref[...] = jnp.dot(h, w_ref[...],
                           preferred_element_type=jnp.float32) * dinv


def _tc_mid(p, g, dinv, b, Wn):
    return pl.pallas_call(
        _tc_mid_body,
        grid=(N // _BLK,),
        in_specs=[_part_spec(0, H), _part_spec(1, H), _row_spec(H),
                  _row_spec(1), _full_spec((1, H)), _full_spec((H, H))],
        out_specs=_row_spec(H),
        out_shape=jax.ShapeDtypeStruct((N, H), jnp.float32),
    )(p, p, g, dinv, b, Wn)


def _tc_final_body(p0_ref, p1_ref, g_ref, dinv_ref, b_ref, wc_ref, bc_ref,
                   wr_ref, br_ref, emb_ref, cls_ref, reg_ref):
    emb = (p0_ref[0] + p1_ref[0] + g_ref[...]) * dinv_ref[...] + b_ref[...]
    emb_ref[...] = emb
    zc = jnp.sum(emb * wc_ref[...], axis=1, keepdims=True) + bc_ref[...]
    cls_ref[...] = jax.nn.sigmoid(zc)
    reg_ref[...] = jnp.sum(emb * wr_ref[...], axis=1, keepdims=True) + br_ref[...]


def _tc_final(p, g, dinv, b3, Wc, bc, Wr, br):
    return pl.pallas_call(
        _tc_final_body,
        grid=(N // _BLK,),
        in_specs=[_part_spec(0, H), _part_spec(1, H), _row_spec(H),
                  _row_spec(1),
                  _full_spec((1, H)), _full_spec((1, H)), _full_spec((1, 1)),
                  _full_spec((1, H)), _full_spec((1, 1))],
        out_specs=[_row_spec(H), _row_spec(1), _row_spec(1)],
        out_shape=[jax.ShapeDtypeStruct((N, H), jnp.float32),
                   jax.ShapeDtypeStruct((N, 1), jnp.float32),
                   jax.ShapeDtypeStruct((N, 1), jnp.float32)],
    )(p, p, g, dinv, b3, Wc, bc, Wr, br)


def kernel(x, edge_index, W1, b1, W2, b2, W3, b3, Wc, bc, Wr, br):
    src = edge_index[0]
    dst = edge_index[1]
    pad = E_PAD - E
    dump = N + jnp.arange(pad, dtype=jnp.int32) % (N_SH - N)
    srcp = jnp.concatenate([src, jnp.zeros((pad,), jnp.int32)])
    dstp = jnp.concatenate([dst, dump])
    srcp = srcp.reshape(NW * CPW, CHUNK)
    dstp = dstp.reshape(NW * CPW, CHUNK)
    ones16 = jnp.ones((CHUNK, 16), jnp.float32)
    zero16 = jnp.zeros((TROWS, 16), jnp.float32)
    zero64 = jnp.zeros((TROWS, H), jnp.float32)

    degw = _sc_degree(dstp, ones16, zero16)
    g1, dinv = _tc_prep(x, W1, degw)
    p = _sc_aggregate(g1, srcp, dstp, zero64)
    g2 = _tc_mid(p, g1, dinv, b1.reshape(1, H), W2)
    p = _sc_aggregate(g2, srcp, dstp, zero64)
    g3 = _tc_mid(p, g2, dinv, b2.reshape(1, H), W3)
    p = _sc_aggregate(g3, srcp, dstp, zero64)
    emb, cls, reg = _tc_final(p, g3, dinv, b3.reshape(1, H),
                              Wc.reshape(1, H), bc.reshape(1, 1),
                              Wr.reshape(1, H), br.reshape(1, 1))
    return (emb, cls, reg)
